# baseline (speedup 1.0000x reference)
import jax
import jax.numpy as jnp
from jax.experimental import pallas as pl
from jax.experimental.pallas import tpu as pltpu

_TS = 26
_TOT = 126
_REP = 20
_F = _TS + (_TOT - _TS) * _REP
_D = 64
_FLAT_ROWS = _F * _D // 128
_S_ROWS = _TS * _D // 128
_T_ROWS = (_TOT - _TS) * _D // 128
_BB = 8


def _body(tbl_s_ref, tbl_t_ref, x_ref, o_ref, bias_ref):
    @pl.when(pl.program_id(0) == 0)
    def _init():
        bias_ref[0:_S_ROWS] = tbl_s_ref[...]
        ts = tbl_t_ref[...]
        for r in range(_REP):
            lo = _S_ROWS + r * _T_ROWS
            bias_ref[lo:lo + _T_ROWS] = ts

    o_ref[...] = x_ref[...] + bias_ref[...][None, :, :]


def kernel(X, table):
    B = X.shape[0]
    x_flat = X.reshape(B, _FLAT_ROWS, 128)
    tbl_s = table[:_TS].reshape(_S_ROWS, 128)
    tbl_t = table[_TS:].reshape(_T_ROWS, 128)

    out = pl.pallas_call(
        _body,
        grid=(B // _BB,),
        in_specs=[
            pl.BlockSpec((_S_ROWS, 128), lambda i: (0, 0)),
            pl.BlockSpec((_T_ROWS, 128), lambda i: (0, 0)),
            pl.BlockSpec((_BB, _FLAT_ROWS, 128), lambda i: (i, 0, 0)),
        ],
        out_specs=pl.BlockSpec((_BB, _FLAT_ROWS, 128), lambda i: (i, 0, 0)),
        out_shape=jax.ShapeDtypeStruct((B, _FLAT_ROWS, 128), X.dtype),
        scratch_shapes=[pltpu.VMEM((_FLAT_ROWS, 128), jnp.float32)],
    )(tbl_s, tbl_t, x_flat)
    return out.reshape(X.shape)
